# baseline (speedup 1.0000x reference)
import functools

import jax
import jax.numpy as jnp
from jax import lax
from jax.experimental import pallas as pl
from jax.experimental.pallas import tpu as pltpu
from jax.experimental.pallas import tpu_sc as plsc

V = 100000
D = 64
S = 200
B = 4096
OUT = 2

VP = 102400
HALFV = VP // 2
BV = 25600
NG = 16
BG = B // NG
NJ = BG // 16
NCHUNK = 5
CS = S // NCHUNK
TWC = HALFV // NG


def _project_table(w, table_t, b):
    def body(w_ref, tlo_ref, thi_ref, b_ref, tw0_ref, tw1_ref):
        tlo = tlo_ref[...]
        thi = thi_ref[...]
        col = pl.program_id(0) * BV + lax.broadcasted_iota(jnp.int32, (1, BV), 1)
        for c, out_ref in ((0, tw0_ref), (1, tw1_ref)):
            lo = lax.dot_general(
                w_ref[c:c + 1, :], tlo, (((1,), (0,)), ((), ())),
                preferred_element_type=jnp.float32)
            lo = jnp.where(col == 0, 0.0, lo) + b_ref[c]
            hi = lax.dot_general(
                w_ref[c:c + 1, :], thi, (((1,), (0,)), ((), ())),
                preferred_element_type=jnp.float32) + b_ref[c]
            lo16 = lax.convert_element_type(
                lax.bitcast_convert_type(
                    lax.convert_element_type(lo, jnp.bfloat16), jnp.uint16),
                jnp.uint32)
            hi16 = lax.convert_element_type(
                lax.bitcast_convert_type(
                    lax.convert_element_type(hi, jnp.bfloat16), jnp.uint16),
                jnp.uint32)
            word = lax.bitcast_convert_type(
                jnp.left_shift(hi16, 16) | lo16, jnp.int32)
            out_ref[...] = jnp.reshape(word, (BV,))

    return pl.pallas_call(
        body,
        grid=(HALFV // BV,),
        in_specs=[
            pl.BlockSpec((OUT, D), lambda i: (0, 0)),
            pl.BlockSpec((D, BV), lambda i: (0, i)),
            pl.BlockSpec((D, BV), lambda i: (0, i + HALFV // BV)),
            pl.BlockSpec(memory_space=pltpu.SMEM),
        ],
        out_specs=[
            pl.BlockSpec((BV,), lambda i: (i,)),
            pl.BlockSpec((BV,), lambda i: (i,)),
        ],
        out_shape=[
            jax.ShapeDtypeStruct((HALFV,), jnp.int32),
            jax.ShapeDtypeStruct((HALFV,), jnp.int32),
        ],
        compiler_params=pltpu.CompilerParams(
            dimension_semantics=("arbitrary",)),
    )(w, table_t, table_t, b)


def _gather_avg(tw0, tw1, text):
    mesh = plsc.VectorSubcoreMesh(core_axis_name="c", subcore_axis_name="s")

    @functools.partial(
        pl.kernel,
        mesh=mesh,
        compiler_params=pltpu.CompilerParams(
            needs_layout_passes=False, disable_bounds_checks=True),
        out_type=jax.ShapeDtypeStruct((OUT * B,), jnp.float32),
        scratch_types=[
            pltpu.VMEM((HALFV,), jnp.int32),
            pltpu.VMEM_SHARED((HALFV,), jnp.int32),
            pltpu.VMEM((2, CS, BG), jnp.int32),
            pltpu.VMEM((BG,), jnp.float32),
            pltpu.SemaphoreType.DMA,
            pltpu.SemaphoreType.DMA,
        ],
    )
    def k(tw0_hbm, tw1_hbm, text_hbm, out_hbm,
          tw_v, tw_sh, text_v, acc_v, sem0, sem1):
        c = lax.axis_index("c")
        g = lax.axis_index("s")
        goff = pl.multiple_of(g * BG, 128)
        sems = (sem0, sem1)
        cps = [None, None]
        cps[0] = pltpu.async_copy(
            text_hbm.at[pl.ds(0, CS), pl.ds(goff, BG)], text_v.at[0], sem0)

        seg = pl.multiple_of(g * TWC, 128)

        @pl.when(c == 0)
        def _():
            pltpu.sync_copy(tw0_hbm.at[pl.ds(seg, TWC)],
                            tw_v.at[pl.ds(seg, TWC)])

        @pl.when(c == 1)
        def _():
            pltpu.sync_copy(tw1_hbm.at[pl.ds(seg, TWC)],
                            tw_v.at[pl.ds(seg, TWC)])

        pltpu.sync_copy(tw_v.at[pl.ds(seg, TWC)], tw_sh.at[pl.ds(seg, TWC)])
        plsc.subcore_barrier()
        pltpu.sync_copy(tw_sh, tw_v)

        himask = jnp.full((16,), -65536, jnp.int32)
        acc = tuple(jnp.zeros((16,), jnp.float32) for _ in range(NJ))
        for kk in range(NCHUNK):
            buf = kk % 2
            cps[buf].wait()
            if kk + 1 < NCHUNK:
                cps[1 - buf] = pltpu.async_copy(
                    text_hbm.at[pl.ds((kk + 1) * CS, CS), pl.ds(goff, BG)],
                    text_v.at[1 - buf], sems[1 - buf])

            def s_body(si, carry):
                out = []
                for j in range(NJ):
                    idx = text_v[buf, si, pl.ds(16 * j, 16)]
                    is_hi = idx >= HALFV
                    word = plsc.load_gather(
                        tw_v, [jnp.where(is_hi, idx - HALFV, idx)])
                    bits = jnp.where(is_hi, word & himask,
                                     jnp.left_shift(word, 16))
                    out.append(carry[j] + plsc.bitcast(bits, jnp.float32))
                return tuple(out)

            acc = plsc.parallel_loop(0, CS, unroll=2, carry=acc)(s_body)

        for j in range(NJ):
            acc_v[pl.ds(16 * j, 16)] = acc[j] * (1.0 / S)
        pltpu.sync_copy(
            acc_v, out_hbm.at[pl.ds(pl.multiple_of(c * B + g * BG, 128), BG)])

    return k(tw0, tw1, text)


def kernel(text, table, W, b):
    tw0, tw1 = _project_table(W, table.T, b)
    out_flat = _gather_avg(tw0, tw1, text)
    return out_flat.reshape(OUT, B).T


# speedup vs baseline: 1.1833x; 1.1833x over previous
import functools

import jax
import jax.numpy as jnp
from jax import lax
from jax.experimental import pallas as pl
from jax.experimental.pallas import tpu as pltpu
from jax.experimental.pallas import tpu_sc as plsc

V = 100000
D = 64
S = 200
B = 4096
OUT = 2

VP = 102400
BV = 25600
NG = 16
BG = B // NG
NJ = BG // 16
NCHUNK = 5
CS = S // NCHUNK
TWC = VP // NG


def _project_table(w, table_t, b):
    def body(w_ref, t_ref, b_ref, tw0_ref, tw1_ref):
        t = t_ref[...]
        col = pl.program_id(0) * BV + lax.broadcasted_iota(jnp.int32, (1, BV), 1)
        for c, out_ref in ((0, tw0_ref), (1, tw1_ref)):
            res = lax.dot_general(
                w_ref[c:c + 1, :], t, (((1,), (0,)), ((), ())),
                preferred_element_type=jnp.float32)
            res = jnp.where(col == 0, 0.0, res) + b_ref[c]
            out_ref[...] = jnp.reshape(res, (BV,))

    return pl.pallas_call(
        body,
        grid=(VP // BV,),
        in_specs=[
            pl.BlockSpec((OUT, D), lambda i: (0, 0)),
            pl.BlockSpec((D, BV), lambda i: (0, i)),
            pl.BlockSpec(memory_space=pltpu.SMEM),
        ],
        out_specs=[
            pl.BlockSpec((BV,), lambda i: (i,)),
            pl.BlockSpec((BV,), lambda i: (i,)),
        ],
        out_shape=[
            jax.ShapeDtypeStruct((VP,), jnp.float32),
            jax.ShapeDtypeStruct((VP,), jnp.float32),
        ],
        compiler_params=pltpu.CompilerParams(
            dimension_semantics=("arbitrary",)),
    )(w, table_t, b)


def _gather_avg(tw0, tw1, text):
    mesh = plsc.VectorSubcoreMesh(core_axis_name="c", subcore_axis_name="s")

    @functools.partial(
        pl.kernel,
        mesh=mesh,
        compiler_params=pltpu.CompilerParams(
            needs_layout_passes=False, disable_bounds_checks=True),
        out_type=jax.ShapeDtypeStruct((OUT * B,), jnp.float32),
        scratch_types=[
            pltpu.VMEM((VP,), jnp.float32),
            pltpu.VMEM_SHARED((VP,), jnp.float32),
            pltpu.VMEM((2, CS, BG), jnp.int32),
            pltpu.VMEM((BG,), jnp.float32),
            pltpu.SemaphoreType.DMA,
            pltpu.SemaphoreType.DMA,
        ],
    )
    def k(tw0_hbm, tw1_hbm, text_hbm, out_hbm,
          tw_v, tw_sh, text_v, acc_v, sem0, sem1):
        c = lax.axis_index("c")
        g = lax.axis_index("s")
        goff = pl.multiple_of(g * BG, 128)
        sems = (sem0, sem1)
        cps = [None, None]
        cps[0] = pltpu.async_copy(
            text_hbm.at[pl.ds(0, CS), pl.ds(goff, BG)], text_v.at[0], sem0)

        seg = pl.multiple_of(g * TWC, 128)

        @pl.when(c == 0)
        def _():
            pltpu.sync_copy(tw0_hbm.at[pl.ds(seg, TWC)],
                            tw_v.at[pl.ds(seg, TWC)])

        @pl.when(c == 1)
        def _():
            pltpu.sync_copy(tw1_hbm.at[pl.ds(seg, TWC)],
                            tw_v.at[pl.ds(seg, TWC)])

        pltpu.sync_copy(tw_v.at[pl.ds(seg, TWC)], tw_sh.at[pl.ds(seg, TWC)])
        plsc.subcore_barrier()
        pltpu.sync_copy(tw_sh, tw_v)

        acc = tuple(jnp.zeros((16,), jnp.float32) for _ in range(NJ))
        for kk in range(NCHUNK):
            buf = kk % 2
            cps[buf].wait()
            if kk + 1 < NCHUNK:
                cps[1 - buf] = pltpu.async_copy(
                    text_hbm.at[pl.ds((kk + 1) * CS, CS), pl.ds(goff, BG)],
                    text_v.at[1 - buf], sems[1 - buf])

            def s_body(si, carry):
                out = []
                for j in range(NJ):
                    idx = text_v[buf, si, pl.ds(16 * j, 16)]
                    vals = plsc.load_gather(tw_v, [idx])
                    out.append(carry[j] + vals)
                return tuple(out)

            acc = plsc.parallel_loop(0, CS, unroll=2, carry=acc)(s_body)

        for j in range(NJ):
            acc_v[pl.ds(16 * j, 16)] = acc[j] * (1.0 / S)
        pltpu.sync_copy(
            acc_v, out_hbm.at[pl.ds(pl.multiple_of(c * B + g * BG, 128), BG)])

    return k(tw0, tw1, text)


def kernel(text, table, W, b):
    tw0, tw1 = _project_table(W, table.T, b)
    out_flat = _gather_avg(tw0, tw1, text)
    return out_flat.reshape(OUT, B).T
